# baseline (speedup 1.0000x reference)
import jax
import jax.numpy as jnp
from jax.experimental import pallas as pl

_TB = 16
_SB = 64
_C = 256
_TP = 64
_SP = 1024
_W = 32
_SIGMA2 = 4.0
_BLK = 4
_PK = _BLK // 2


def _xcorr_kernel(t_ref, s_ref, o_ref):
    t = t_ref[...]
    s = s_ref[...]
    s_e = s[:, 0]
    s_o = s[:, 1]
    dn = (((2,), (2,)), ((0,), (0,)))
    m_e = jax.lax.dot_general(s_e, t, dn, preferred_element_type=jnp.float32)
    m_o = jax.lax.dot_general(s_o, t, dn, preferred_element_type=jnp.float32)
    lane = jax.lax.broadcasted_iota(jnp.int32, (1, 1, 2 * _TP), 2)
    emask = lane < _TP
    xc = jnp.where(emask, m_e, m_o)

    mt2 = jnp.sum(t * t, axis=2, keepdims=True)
    mt2 = jnp.transpose(mt2, (0, 2, 1))
    ms2 = jnp.sum(s * s, axis=3, keepdims=True)
    ms2p = jnp.where(emask, ms2[:, 0], ms2[:, 1])
    norm = jnp.sqrt(mt2) * jnp.sqrt(ms2p) + 1e-8
    xc = xc / norm

    m = jnp.max(xc, axis=1, keepdims=True)
    j = jax.lax.broadcasted_iota(jnp.int32, (1, _SP, 2 * _TP), 1)
    idx = jnp.min(jnp.where(xc == m, j, _SP), axis=1, keepdims=True)

    fidx = idx.astype(jnp.float32)
    py = fidx * (1.0 / _W)
    px = (idx & (_W - 1)).astype(jnp.float32)

    y = (j >> 5).astype(jnp.float32)
    x = (j & (_W - 1)).astype(jnp.float32)
    dy = y - py
    dx = x - px
    g = jnp.exp((-0.5 / _SIGMA2) * (dy * dy + dx * dx))
    res = xc * g
    for p in range(_PK):
        o_ref[2 * p] = res[p, :, 0:_TP]
        o_ref[2 * p + 1] = res[p, :, _TP:2 * _TP]


def kernel(template, search):
    t = template.transpose(0, 2, 3, 1).reshape(_TB // 2, 2 * _TP, _C)
    s = search.transpose(0, 2, 3, 1).reshape(_SB // 2, 2, _SP, _C)

    out = pl.pallas_call(
        _xcorr_kernel,
        grid=(_SB // _BLK,),
        in_specs=[
            pl.BlockSpec((_PK, 2 * _TP, _C), lambda b: (b % (_TB // _BLK), 0, 0)),
            pl.BlockSpec((_PK, 2, _SP, _C), lambda b: (b, 0, 0, 0)),
        ],
        out_specs=pl.BlockSpec((_BLK, _SP, _TP), lambda b: (b, 0, 0)),
        out_shape=jax.ShapeDtypeStruct((_SB, _SP, _TP), jnp.float32),
    )(t, s)

    return out.reshape(_SB, _W, _W, _TP).transpose(0, 3, 1, 2)


# speedup vs baseline: 1.1094x; 1.1094x over previous
import jax
import jax.numpy as jnp
from jax.experimental import pallas as pl
from jax.experimental.pallas import tpu as pltpu

_TB = 16
_SB = 64
_C = 256
_TP = 64
_SP = 1024
_W = 32
_SIGMA2 = 4.0
_BLK = 8
_PK = _BLK // 2


def _xcorr_kernel(t_ref, s_ref, o_ref):
    t = t_ref[...]
    s = s_ref[...]
    s_e = s[:, 0]
    s_o = s[:, 1]
    dn = (((2,), (2,)), ((0,), (0,)))
    m_e = jax.lax.dot_general(s_e, t, dn, preferred_element_type=jnp.float32)
    m_o = jax.lax.dot_general(s_o, t, dn, preferred_element_type=jnp.float32)
    lane = jax.lax.broadcasted_iota(jnp.int32, (1, 1, 2 * _TP), 2)
    emask = lane < _TP
    xc = jnp.where(emask, m_e, m_o)

    mt2 = jnp.sum(t * t, axis=2, keepdims=True)
    mt2 = jnp.transpose(mt2, (0, 2, 1))
    ms2 = jnp.sum(s * s, axis=3, keepdims=True)
    ms2p = jnp.where(emask, ms2[:, 0], ms2[:, 1])
    norm = jnp.sqrt(mt2) * jnp.sqrt(ms2p) + 1e-8
    xc = xc / norm

    m = jnp.max(xc, axis=1, keepdims=True)
    j = jax.lax.broadcasted_iota(jnp.int32, (1, _SP, 2 * _TP), 1)
    idx = jnp.min(jnp.where(xc == m, j, _SP), axis=1, keepdims=True)

    fidx = idx.astype(jnp.float32)
    py = fidx * (1.0 / _W)
    px = (idx & (_W - 1)).astype(jnp.float32)

    y = (j >> 5).astype(jnp.float32)
    x = (j & (_W - 1)).astype(jnp.float32)
    dy = y - py
    dx = x - px
    g = jnp.exp((-0.5 / _SIGMA2) * (dy * dy + dx * dx))
    res = xc * g
    for p in range(_PK):
        o_ref[2 * p] = res[p, :, 0:_TP]
        o_ref[2 * p + 1] = res[p, :, _TP:2 * _TP]


def kernel(template, search):
    t = template.transpose(0, 2, 3, 1).reshape(_TB // 2, 2 * _TP, _C)
    s = search.transpose(0, 2, 3, 1).reshape(_SB // 2, 2, _SP, _C)

    out = pl.pallas_call(
        _xcorr_kernel,
        grid=(_SB // _BLK,),
        in_specs=[
            pl.BlockSpec((_PK, 2 * _TP, _C), lambda b: (b % 2, 0, 0)),
            pl.BlockSpec((_PK, 2, _SP, _C), lambda b: (b, 0, 0, 0)),
        ],
        out_specs=pl.BlockSpec((_BLK, _SP, _TP), lambda b: (b, 0, 0)),
        out_shape=jax.ShapeDtypeStruct((_SB, _SP, _TP), jnp.float32),
        compiler_params=pltpu.CompilerParams(
            dimension_semantics=("parallel",)),
    )(t, s)

    return out.reshape(_SB, _W, _W, _TP).transpose(0, 3, 1, 2)


# speedup vs baseline: 1.1125x; 1.0028x over previous
import jax
import jax.numpy as jnp
from jax.experimental import pallas as pl

_TB = 16
_SB = 64
_C = 256
_TP = 64
_SP = 1024
_W = 32
_SIGMA2 = 4.0
_BLK = 8
_PK = _BLK // 2


def _xcorr_kernel(t_ref, s_ref, o_ref):
    t = t_ref[...]
    s = s_ref[...]
    s_e = s[:, 0]
    s_o = s[:, 1]
    dn = (((2,), (2,)), ((0,), (0,)))
    m_e = jax.lax.dot_general(s_e, t, dn, preferred_element_type=jnp.float32)
    m_o = jax.lax.dot_general(s_o, t, dn, preferred_element_type=jnp.float32)
    lane = jax.lax.broadcasted_iota(jnp.int32, (1, 1, 2 * _TP), 2)
    emask = lane < _TP
    xc = jnp.where(emask, m_e, m_o)

    mt2 = jnp.sum(t * t, axis=2, keepdims=True)
    mt2 = jnp.transpose(mt2, (0, 2, 1))
    ms2 = jnp.sum(s * s, axis=3, keepdims=True)
    ms2p = jnp.where(emask, ms2[:, 0], ms2[:, 1])
    norm = jnp.sqrt(mt2) * jnp.sqrt(ms2p) + 1e-8
    xc = xc / norm

    m = jnp.max(xc, axis=1, keepdims=True)
    j = jax.lax.broadcasted_iota(jnp.int32, (1, _SP, 2 * _TP), 1)
    idx = jnp.min(jnp.where(xc == m, j, _SP), axis=1, keepdims=True)

    fidx = idx.astype(jnp.float32)
    py = fidx * (1.0 / _W)
    px = (idx & (_W - 1)).astype(jnp.float32)

    y = (j >> 5).astype(jnp.float32)
    x = (j & (_W - 1)).astype(jnp.float32)
    dy = y - py
    dx = x - px
    g = jnp.exp((-0.5 / _SIGMA2) * (dy * dy + dx * dx))
    res = xc * g
    for p in range(_PK):
        o_ref[2 * p] = res[p, :, 0:_TP]
        o_ref[2 * p + 1] = res[p, :, _TP:2 * _TP]


def kernel(template, search):
    t = template.transpose(0, 2, 3, 1).reshape(_TB // 2, 2 * _TP, _C)
    s = search.transpose(0, 2, 3, 1).reshape(_SB // 2, 2, _SP, _C)

    out = pl.pallas_call(
        _xcorr_kernel,
        grid=(_SB // _BLK,),
        in_specs=[
            pl.BlockSpec((_PK, 2 * _TP, _C), lambda b: (b % 2, 0, 0)),
            pl.BlockSpec((_PK, 2, _SP, _C), lambda b: (b, 0, 0, 0)),
        ],
        out_specs=pl.BlockSpec((_BLK, _SP, _TP), lambda b: (b, 0, 0)),
        out_shape=jax.ShapeDtypeStruct((_SB, _SP, _TP), jnp.float32),
    )(t, s)

    return out.reshape(_SB, _W, _W, _TP).transpose(0, 3, 1, 2)
